# baseline (speedup 1.0000x reference)
import functools

import jax
import jax.numpy as jnp
import numpy as np
from jax import lax
from jax.experimental import pallas as pl
from jax.experimental.pallas import tpu as pltpu
from jax.experimental.pallas import tpu_sc as plsc

_FIELD = 26
_FEATURE = 100000
_EMB = 16
_BATCH = 16384
_F2FEAT = [(i * 3571) % _FEATURE for i in range(_FIELD)]

_NC = 2
_NS = 16
_NW = _NC * _NS

_SPLITS = (10240, 6144)
_CHUNK = 1664
_ROWS_C = _CHUNK // _FIELD
_WPC = 2 * _ROWS_C


def _consts(bs):
    b = np.arange(bs, dtype=np.int64)
    f = np.arange(_FIELD, dtype=np.int64)
    scat_e = np.asarray(
        (b[:, None] // 8) * 256 + ((16 * f[None, :]) // 128) * 64
        + (b[:, None] % 8) * 8 + ((16 * f[None, :]) % 128) // 16,
        dtype=np.int32).reshape(-1)
    wbase = (b // 8) * 256 + 3 * 64 + (b % 8) * 8 + 2
    scat_w = np.stack([wbase, wbase + 1], axis=1).astype(np.int32).reshape(-1)
    return scat_e, scat_w


_s = np.arange(_CHUNK, dtype=np.int64)
_WDST_R = np.asarray((_s // _FIELD) * 2 + (_s % _FIELD) // 16, dtype=np.int32)
_WDST_C = np.asarray((_s % _FIELD) % 16, dtype=np.int32)


def _make_sc_gather(bs):
  per_w = bs * _FIELD // _NW
  nchunk = per_w // _CHUNK
  rows_w = bs // _NW

  def _sc_gather(emb_hbm, fmw_hbm, ids_hbm, se_hbm, sw_hbm, wr_hbm, wc_hbm,
                 out_hbm, idx_v, rows_v, w_v, se_v, sw_v, wp_v, wr_v, wc_v,
                 fmw_sh, sem_r0, sem_w0, sem_r1, sem_w1,
                 sem_se0, sem_sw0, sem_se1, sem_sw1):
    sid = lax.axis_index("s")
    wid = sid * _NC + lax.axis_index("c")
    gsems = ((sem_r0, sem_w0), (sem_r1, sem_w1))
    ssems = ((sem_se0, sem_sw0), (sem_se1, sem_sw1))

    @pl.when(sid == 0)
    def _():
        pltpu.sync_copy(fmw_hbm, fmw_sh)

    pltpu.sync_copy(wr_hbm, wr_v)
    pltpu.sync_copy(wc_hbm, wc_v)
    zeros = jnp.zeros((16,), jnp.float32)
    for i in range(2):
        for k in range(_WPC):
            wp_v[i][k, :] = zeros

    plsc.subcore_barrier()

    def start(c):
        i = c % 2
        off = pl.multiple_of(wid * per_w + c * _CHUNK, _CHUNK)
        woff = pl.multiple_of(wid * (2 * rows_w) + c * _WPC, _WPC)
        pltpu.sync_copy(ids_hbm.at[pl.ds(off, _CHUNK)], idx_v[i])
        pltpu.sync_copy(se_hbm.at[pl.ds(off, _CHUNK)], se_v[i])
        pltpu.sync_copy(sw_hbm.at[pl.ds(woff, _WPC)], sw_v[i])
        cp_r = pltpu.async_copy(emb_hbm.at[idx_v[i]], rows_v[i], gsems[i][0])
        cp_w = pltpu.async_copy(fmw_sh.at[idx_v[i]], w_v[i], gsems[i][1])
        return cp_r, cp_w

    def drain(c, cps):
        i = c % 2
        cps[0].wait()
        cps[1].wait()
        for k in range(_CHUNK // 16):
            sl = pl.ds(k * 16, 16)
            plsc.store_scatter(wp_v[i], [wr_v[sl], wc_v[sl]], w_v[i][sl])
        cp_se = pltpu.async_copy(rows_v[i], out_hbm.at[se_v[i]], ssems[i][0])
        cp_sw = pltpu.async_copy(wp_v[i], out_hbm.at[sw_v[i]], ssems[i][1])
        return cp_se, cp_sw

    outstanding = [None, None]
    prev = start(0)
    for c in range(1, nchunk):
        if outstanding[c % 2] is not None:
            outstanding[c % 2][0].wait()
            outstanding[c % 2][1].wait()
            outstanding[c % 2] = None
        cur = start(c)
        outstanding[(c - 1) % 2] = drain(c - 1, prev)
        prev = cur
    for i in range(2):
        if outstanding[i] is not None:
            outstanding[i][0].wait()
            outstanding[i][1].wait()
    last = drain(nchunk - 1, prev)
    last[0].wait()
    last[1].wait()

  return _sc_gather


def _gather_call(emb, fmw, ids, bs):
    scat_e, scat_w = _consts(bs)
    call = functools.partial(
        pl.kernel,
        out_type=jax.ShapeDtypeStruct((bs * 32, 16), jnp.float32),
        mesh=plsc.VectorSubcoreMesh(core_axis_name="c", subcore_axis_name="s"),
        scratch_types=[
            [pltpu.VMEM((_CHUNK,), jnp.int32)] * 2,
            [pltpu.VMEM((_CHUNK, _EMB), jnp.float32)] * 2,
            [pltpu.VMEM((_CHUNK,), jnp.float32)] * 2,
            [pltpu.VMEM((_CHUNK,), jnp.int32)] * 2,
            [pltpu.VMEM((_WPC,), jnp.int32)] * 2,
            [pltpu.VMEM((_WPC, 16), jnp.float32)] * 2,
            pltpu.VMEM((_CHUNK,), jnp.int32),
            pltpu.VMEM((_CHUNK,), jnp.int32),
            pltpu.VMEM_SHARED((_FEATURE,), jnp.float32),
        ] + [pltpu.SemaphoreType.DMA] * 8,
        compiler_params=pltpu.CompilerParams(use_tc_tiling_on_sc=False, needs_layout_passes=False),
    )(_make_sc_gather(bs))
    return call(emb, fmw, ids, jnp.asarray(scat_e), jnp.asarray(scat_w),
                jnp.asarray(_WDST_R), jnp.asarray(_WDST_C))


_BB = 1024


def _tc_fused(xt_ref, vals_ref, a_ref, w0_ref, b0_ref, w1_ref,
              b1_ref, owt_ref, bias_ref, out_ref):
    x0 = xt_ref[:, 0].reshape(_BB, 128)
    x1 = xt_ref[:, 1].reshape(_BB, 128)
    x2 = xt_ref[:, 2].reshape(_BB, 128)
    x3 = xt_ref[:, 3].reshape(_BB, 128)
    w0 = w0_ref[...]
    x01 = jnp.concatenate([x0, x1], axis=1)
    x23 = jnp.concatenate([x2, x3[:, :64]], axis=1)
    h = jnp.dot(x01, w0[0:256], preferred_element_type=jnp.float32)
    h = h + jnp.dot(x23, w0[256:448], preferred_element_type=jnp.float32)
    h = jnp.maximum(h + b0_ref[...], 0.0)
    h = jnp.dot(h, w1_ref[...], preferred_element_type=jnp.float32)
    h = jnp.maximum(h + b1_ref[...], 0.0)
    deep = jnp.sum(h * owt_ref[...], axis=1, keepdims=True)

    vals = vals_ref[...]
    w = x3[:, 32:32 + _FIELD]
    lin = jnp.sum(w * vals, axis=1, keepdims=True)

    a = a_ref[...]
    t = jnp.dot(vals, a, preferred_element_type=jnp.float32)
    n2 = jnp.sum(a * a, axis=1, keepdims=True)
    diag = jnp.dot(vals * vals, n2, preferred_element_type=jnp.float32)
    inter = 0.5 * (jnp.sum(t * t, axis=1, keepdims=True) - diag)

    out = jax.nn.sigmoid(deep + lin + inter + bias_ref[...])
    out_ref[...] = out


def _tc_call(xt4, vals, a, w0p, b0, w1, b1, owt, bias, row0, bs):
    grid = bs // _BB
    rep = lambda i: (0, 0)
    voff = row0 // _BB
    return pl.pallas_call(
        _tc_fused,
        grid=(grid,),
        in_specs=[
            pl.BlockSpec((_BB // 8, 4, 8, 128), lambda i: (i, 0, 0, 0)),
            pl.BlockSpec((_BB, _FIELD), lambda i: (i + voff, 0)),
            pl.BlockSpec((_FIELD, _EMB), rep),
            pl.BlockSpec((448, 512), rep),
            pl.BlockSpec((1, 512), rep),
            pl.BlockSpec((512, 256), rep),
            pl.BlockSpec((1, 256), rep),
            pl.BlockSpec((1, 256), rep),
            pl.BlockSpec((1, 1), rep),
        ],
        out_specs=pl.BlockSpec((_BB, 1), lambda i: (i, 0)),
        out_shape=jax.ShapeDtypeStruct((bs, 1), jnp.float32),
    )(xt4, vals, a, w0p, b0, w1, b1, owt, bias)


def kernel(feat_ids, feat_vals, FM_W, FM_V, FM_B, embedding,
           deepW0, deepB0, deepW1, deepB1, outW, outB):
    a = jnp.stack([FM_V[i, _F2FEAT[i], 0, :] for i in range(_FIELD)])
    bias = (FM_B[0] + outB[0]).reshape(1, 1)
    w0p = jnp.concatenate(
        [deepW0, jnp.zeros((32, 512), jnp.float32)], axis=0)
    b0 = deepB0.reshape(1, -1)
    b1 = deepB1.reshape(1, -1)
    owt = outW.reshape(1, -1)
    ids_flat = feat_ids.reshape(-1)
    parts = []
    row0 = 0
    for bs in _SPLITS:
        ids_s = ids_flat[row0 * _FIELD:(row0 + bs) * _FIELD]
        outT = _gather_call(embedding, FM_W, ids_s, bs)
        xt4 = outT.reshape(bs // 8, 4, 8, 128)
        parts.append(_tc_call(xt4, feat_vals, a, w0p, b0, deepW1, b1,
                              owt, bias, row0, bs))
        row0 += bs
    return jnp.concatenate(parts, axis=0).reshape(-1)


# speedup vs baseline: 1.0563x; 1.0563x over previous
import functools

import jax
import jax.numpy as jnp
import numpy as np
from jax import lax
from jax.experimental import pallas as pl
from jax.experimental.pallas import tpu as pltpu
from jax.experimental.pallas import tpu_sc as plsc

_FIELD = 26
_FEATURE = 100000
_EMB = 16
_BATCH = 16384
_F2FEAT = [(i * 3571) % _FEATURE for i in range(_FIELD)]

_NC = 2
_NS = 16
_NW = _NC * _NS

_SPLITS = (8192, 8192)
_CHUNK = 1664
_ROWS_C = _CHUNK // _FIELD
_WPC = 2 * _ROWS_C


def _consts(bs):
    b = np.arange(bs, dtype=np.int64)
    f = np.arange(_FIELD, dtype=np.int64)
    scat_e = np.asarray(
        (b[:, None] // 8) * 256 + ((16 * f[None, :]) // 128) * 64
        + (b[:, None] % 8) * 8 + ((16 * f[None, :]) % 128) // 16,
        dtype=np.int32).reshape(-1)
    wbase = (b // 8) * 256 + 3 * 64 + (b % 8) * 8 + 2
    scat_w = np.stack([wbase, wbase + 1], axis=1).astype(np.int32).reshape(-1)
    return scat_e, scat_w


_s = np.arange(_CHUNK, dtype=np.int64)
_WDST_R = np.asarray((_s // _FIELD) * 2 + (_s % _FIELD) // 16, dtype=np.int32)
_WDST_C = np.asarray((_s % _FIELD) % 16, dtype=np.int32)


def _make_sc_gather(bs):
  per_w = bs * _FIELD // _NW
  nchunk = per_w // _CHUNK
  rows_w = bs // _NW

  def _sc_gather(emb_hbm, fmw_hbm, ids_hbm, se_hbm, sw_hbm, wr_hbm, wc_hbm,
                 out_hbm, idx_v, rows_v, w_v, se_v, sw_v, wp_v, wr_v, wc_v,
                 fmw_sh, sem_r0, sem_w0, sem_r1, sem_w1,
                 sem_se0, sem_sw0, sem_se1, sem_sw1):
    sid = lax.axis_index("s")
    wid = sid * _NC + lax.axis_index("c")
    gsems = ((sem_r0, sem_w0), (sem_r1, sem_w1))
    ssems = ((sem_se0, sem_sw0), (sem_se1, sem_sw1))

    @pl.when(sid == 0)
    def _():
        pltpu.sync_copy(fmw_hbm, fmw_sh)

    pltpu.sync_copy(wr_hbm, wr_v)
    pltpu.sync_copy(wc_hbm, wc_v)
    zeros = jnp.zeros((16,), jnp.float32)
    for i in range(2):
        for k in range(_WPC):
            wp_v[i][k, :] = zeros

    plsc.subcore_barrier()

    def start(c):
        i = c % 2
        off = pl.multiple_of(wid * per_w + c * _CHUNK, _CHUNK)
        woff = pl.multiple_of(wid * (2 * rows_w) + c * _WPC, _WPC)
        pltpu.sync_copy(ids_hbm.at[pl.ds(off, _CHUNK)], idx_v[i])
        pltpu.sync_copy(se_hbm.at[pl.ds(off, _CHUNK)], se_v[i])
        pltpu.sync_copy(sw_hbm.at[pl.ds(woff, _WPC)], sw_v[i])
        cp_r = pltpu.async_copy(emb_hbm.at[idx_v[i]], rows_v[i], gsems[i][0])
        cp_w = pltpu.async_copy(fmw_sh.at[idx_v[i]], w_v[i], gsems[i][1])
        return cp_r, cp_w

    def drain(c, cps):
        i = c % 2
        cps[0].wait()
        cps[1].wait()
        for k in range(_CHUNK // 16):
            sl = pl.ds(k * 16, 16)
            plsc.store_scatter(wp_v[i], [wr_v[sl], wc_v[sl]], w_v[i][sl])
        cp_se = pltpu.async_copy(rows_v[i], out_hbm.at[se_v[i]], ssems[i][0])
        cp_sw = pltpu.async_copy(wp_v[i], out_hbm.at[sw_v[i]], ssems[i][1])
        return cp_se, cp_sw

    outstanding = [None, None]
    prev = start(0)
    for c in range(1, nchunk):
        if outstanding[c % 2] is not None:
            outstanding[c % 2][0].wait()
            outstanding[c % 2][1].wait()
            outstanding[c % 2] = None
        cur = start(c)
        outstanding[(c - 1) % 2] = drain(c - 1, prev)
        prev = cur
    for i in range(2):
        if outstanding[i] is not None:
            outstanding[i][0].wait()
            outstanding[i][1].wait()
    last = drain(nchunk - 1, prev)
    last[0].wait()
    last[1].wait()

  return _sc_gather


def _gather_call(emb, fmw, ids, bs):
    scat_e, scat_w = _consts(bs)
    call = functools.partial(
        pl.kernel,
        out_type=jax.ShapeDtypeStruct((bs * 32, 16), jnp.float32),
        mesh=plsc.VectorSubcoreMesh(core_axis_name="c", subcore_axis_name="s"),
        scratch_types=[
            [pltpu.VMEM((_CHUNK,), jnp.int32)] * 2,
            [pltpu.VMEM((_CHUNK, _EMB), jnp.float32)] * 2,
            [pltpu.VMEM((_CHUNK,), jnp.float32)] * 2,
            [pltpu.VMEM((_CHUNK,), jnp.int32)] * 2,
            [pltpu.VMEM((_WPC,), jnp.int32)] * 2,
            [pltpu.VMEM((_WPC, 16), jnp.float32)] * 2,
            pltpu.VMEM((_CHUNK,), jnp.int32),
            pltpu.VMEM((_CHUNK,), jnp.int32),
            pltpu.VMEM_SHARED((_FEATURE,), jnp.float32),
        ] + [pltpu.SemaphoreType.DMA] * 8,
        compiler_params=pltpu.CompilerParams(use_tc_tiling_on_sc=False, needs_layout_passes=False),
    )(_make_sc_gather(bs))
    return call(emb, fmw, ids, jnp.asarray(scat_e), jnp.asarray(scat_w),
                jnp.asarray(_WDST_R), jnp.asarray(_WDST_C))


_BB = 1024


def _tc_fused(xt_ref, vals_ref, a_ref, w0_ref, b0_ref, w1_ref,
              b1_ref, owt_ref, bias_ref, out_ref):
    x0 = xt_ref[:, 0].reshape(_BB, 128)
    x1 = xt_ref[:, 1].reshape(_BB, 128)
    x2 = xt_ref[:, 2].reshape(_BB, 128)
    x3 = xt_ref[:, 3].reshape(_BB, 128)
    w0 = w0_ref[...]
    x01 = jnp.concatenate([x0, x1], axis=1)
    x23 = jnp.concatenate([x2, x3[:, :64]], axis=1)
    h = jnp.dot(x01, w0[0:256], preferred_element_type=jnp.float32)
    h = h + jnp.dot(x23, w0[256:448], preferred_element_type=jnp.float32)
    h = jnp.maximum(h + b0_ref[...], 0.0)
    h = jnp.dot(h, w1_ref[...], preferred_element_type=jnp.float32)
    h = jnp.maximum(h + b1_ref[...], 0.0)
    deep = jnp.sum(h * owt_ref[...], axis=1, keepdims=True)

    vals = vals_ref[...]
    w = x3[:, 32:32 + _FIELD]
    lin = jnp.sum(w * vals, axis=1, keepdims=True)

    a = a_ref[...]
    t = jnp.dot(vals, a, preferred_element_type=jnp.float32)
    n2 = jnp.sum(a * a, axis=1, keepdims=True)
    diag = jnp.dot(vals * vals, n2, preferred_element_type=jnp.float32)
    inter = 0.5 * (jnp.sum(t * t, axis=1, keepdims=True) - diag)

    out = jax.nn.sigmoid(deep + lin + inter + bias_ref[...])
    out_ref[...] = out


def _tc_call(xt4, vals, a, w0p, b0, w1, b1, owt, bias, row0, bs):
    grid = bs // _BB
    rep = lambda i: (0, 0)
    voff = row0 // _BB
    return pl.pallas_call(
        _tc_fused,
        grid=(grid,),
        in_specs=[
            pl.BlockSpec((_BB // 8, 4, 8, 128), lambda i: (i, 0, 0, 0)),
            pl.BlockSpec((_BB, _FIELD), lambda i: (i + voff, 0)),
            pl.BlockSpec((_FIELD, _EMB), rep),
            pl.BlockSpec((448, 512), rep),
            pl.BlockSpec((1, 512), rep),
            pl.BlockSpec((512, 256), rep),
            pl.BlockSpec((1, 256), rep),
            pl.BlockSpec((1, 256), rep),
            pl.BlockSpec((1, 1), rep),
        ],
        out_specs=pl.BlockSpec((_BB, 1), lambda i: (i, 0)),
        out_shape=jax.ShapeDtypeStruct((bs, 1), jnp.float32),
    )(xt4, vals, a, w0p, b0, w1, b1, owt, bias)


def kernel(feat_ids, feat_vals, FM_W, FM_V, FM_B, embedding,
           deepW0, deepB0, deepW1, deepB1, outW, outB):
    a = jnp.stack([FM_V[i, _F2FEAT[i], 0, :] for i in range(_FIELD)])
    bias = (FM_B[0] + outB[0]).reshape(1, 1)
    w0p = jnp.concatenate(
        [deepW0, jnp.zeros((32, 512), jnp.float32)], axis=0)
    b0 = deepB0.reshape(1, -1)
    b1 = deepB1.reshape(1, -1)
    owt = outW.reshape(1, -1)
    parts = []
    row0 = 0
    for bs in _SPLITS:
        ids_s = feat_ids[row0:row0 + bs].reshape(-1)
        outT = _gather_call(embedding, FM_W, ids_s, bs)
        xt4 = outT.reshape(bs // 8, 4, 8, 128)
        parts.append(_tc_call(xt4, feat_vals, a, w0p, b0, deepW1, b1,
                              owt, bias, row0, bs))
        row0 += bs
    return jnp.concatenate(parts, axis=0).reshape(-1)
